# baseline (speedup 1.0000x reference)
import functools

import jax
import jax.numpy as jnp
from jax import lax
from jax.experimental import pallas as pl
from jax.experimental.pallas import tpu as pltpu, tpu_sc as plsc

B = 4096
N = 16384
NNZ = 12300
LANES = 16
CAP_CHUNKS = 1152
CAP = CAP_CHUNKS * LANES

NUM_CORES = 2
NUM_SUBCORES = 16
NW = NUM_CORES * NUM_SUBCORES
ROWS_PER_W = B // NW
PAIRS_PER_W = ROWS_PER_W // 2


def _sc_kernel(x_hbm, w_hbm, b_hbm, vel_hbm, ji_hbm, out_hbm,
               jiref, vref, bias_v, x0, x1, o0, o1,
               sx0, sx1, so0, so1):
    wid = lax.axis_index("s") * NUM_CORES + lax.axis_index("c")
    base_row = wid * ROWS_PER_W

    pltpu.sync_copy(ji_hbm, jiref)
    pltpu.sync_copy(w_hbm, vref)
    pltpu.sync_copy(vel_hbm, x0)
    pltpu.sync_copy(b_hbm, bias_v)

    @plsc.parallel_loop(0, CAP_CHUNKS, unroll=4)
    def _(c):
        s = pl.ds(c * LANES, LANES)
        j = jiref[s] & (N - 1)
        g = plsc.load_gather(x0, [j])
        vref[s] = vref[s] * g

    xbufs, obufs = (x0, x1), (o0, o1)
    xsems, osems = (sx0, sx1), (so0, so1)

    pltpu.async_copy(x_hbm.at[base_row], x0, sx0)

    def pair_body(it, carry):
        for bslot in range(2):
            r = base_row + it * 2 + bslot
            xb, ob = xbufs[bslot], obufs[bslot]
            xs, os_ = xsems[bslot], osems[bslot]

            @pl.when(it * 2 + bslot + 1 < ROWS_PER_W)
            def _():
                pltpu.async_copy(
                    x_hbm.at[r + 1], xbufs[1 - bslot], xsems[1 - bslot])

            @pl.when(it >= 1)
            def _():
                pltpu.make_async_copy(ob, out_hbm.at[r - 2], os_).wait()

            @plsc.parallel_loop(0, N // LANES, unroll=16)
            def _(c):
                s = pl.ds(c * LANES, LANES)
                ob[s] = bias_v[s]

            pltpu.make_async_copy(x_hbm.at[r], xb, xs).wait()

            @plsc.parallel_loop(0, CAP_CHUNKS, unroll=8)
            def _(c):
                s = pl.ds(c * LANES, LANES)
                ji = jiref[s]
                v = vref[s]
                j = ji & (N - 1)
                i = lax.shift_right_logical(ji, 14)
                g = plsc.load_gather(xb, [j])
                plsc.addupdate_scatter(ob, [i], v * g)

            pltpu.async_copy(ob, out_hbm.at[r], os_)
        return carry

    lax.fori_loop(0, PAIRS_PER_W, pair_body, 0)

    pltpu.make_async_copy(o0, out_hbm.at[base_row + ROWS_PER_W - 2], so0).wait()
    pltpu.make_async_copy(o1, out_hbm.at[base_row + ROWS_PER_W - 1], so1).wait()


_mesh = plsc.VectorSubcoreMesh(core_axis_name="c", subcore_axis_name="s")

_call = functools.partial(
    pl.kernel,
    mesh=_mesh,
    out_type=jax.ShapeDtypeStruct((B, N), jnp.float32),
    compiler_params=pltpu.CompilerParams(needs_layout_passes=False),
    scratch_types=[
        pltpu.VMEM((CAP,), jnp.int32),
        pltpu.VMEM((CAP,), jnp.float32),
        pltpu.VMEM((N,), jnp.float32),
        pltpu.VMEM((N,), jnp.float32),
        pltpu.VMEM((N,), jnp.float32),
        pltpu.VMEM((N,), jnp.float32),
        pltpu.VMEM((N,), jnp.float32),
        pltpu.SemaphoreType.DMA,
        pltpu.SemaphoreType.DMA,
        pltpu.SemaphoreType.DMA,
        pltpu.SemaphoreType.DMA,
    ],
)(_sc_kernel)


def kernel(inputs, W3, b, velocity, I, J):
    jb = J & 15
    d = (I - J) & 15
    bucket = d * 16 + jb
    order = jnp.argsort(bucket, stable=True)
    bsort = bucket[order]
    counts = jnp.bincount(bucket, length=256)
    starts = jnp.cumsum(counts) - counts
    rank = jnp.arange(NNZ, dtype=jnp.int32) - starts[bsort].astype(jnp.int32)
    maxd = counts.reshape(16, 16).max(axis=1)
    based = (jnp.cumsum(maxd) - maxd).astype(jnp.int32)
    chunk_idx = based[bsort >> 4] + rank
    slot_cf = chunk_idx * 16 + (bsort & 15)
    fits = maxd.sum() <= CAP_CHUNKS
    slot = jnp.where(fits, slot_cf, order.astype(jnp.int32))
    ji_sorted = (I * N + J)[order]
    w_sorted = W3[order]
    ji_p = jnp.zeros((CAP,), jnp.int32).at[slot].set(
        ji_sorted, unique_indices=True)
    w_p = jnp.zeros((CAP,), jnp.float32).at[slot].set(
        w_sorted, unique_indices=True)
    return _call(inputs, w_p, b, velocity, ji_p)


# speedup vs baseline: 1.1316x; 1.1316x over previous
import functools

import jax
import jax.numpy as jnp
from jax import lax
from jax.experimental import pallas as pl
from jax.experimental.pallas import tpu as pltpu, tpu_sc as plsc

B = 4096
N = 16384
NNZ = 12300
LANES = 16
NNZP = ((NNZ + LANES - 1) // LANES) * LANES
SRC_CHUNKS = NNZP // LANES
CAP_CHUNKS = 1152
CAP = CAP_CHUNKS * LANES

NUM_CORES = 2
NUM_SUBCORES = 16
NW = NUM_CORES * NUM_SUBCORES
ROWS_PER_W = B // NW
PAIRS_PER_W = ROWS_PER_W // 2

def _sc_kernel(x_hbm, w_hbm, b_hbm, vel_hbm, jif_hbm, slotf_hbm, out_hbm,
               jiref, vref, bias_v, x0, x1, o0, o1,
               sx0, sx1, so0, so1):
    wid = lax.axis_index("s") * NUM_CORES + lax.axis_index("c")
    base_row = wid * ROWS_PER_W

    pltpu.sync_copy(jif_hbm, o0.at[pl.ds(0, NNZP)])
    pltpu.sync_copy(slotf_hbm, x1.at[pl.ds(0, NNZP)])
    pltpu.sync_copy(w_hbm, o1.at[pl.ds(0, NNZP)])

    @plsc.parallel_loop(0, CAP_CHUNKS, unroll=8)
    def _(c):
        s = pl.ds(c * LANES, LANES)
        jiref[s] = jnp.zeros((LANES,), jnp.int32)
        vref[s] = jnp.zeros((LANES,), jnp.float32)

    @plsc.parallel_loop(0, SRC_CHUNKS, unroll=4)
    def _(c):
        s = pl.ds(c * LANES, LANES)
        sl = plsc.bitcast(x1[s], jnp.int32)
        jiv = plsc.bitcast(o0[s], jnp.int32)
        plsc.store_scatter(jiref, [sl], jiv)
        plsc.store_scatter(vref, [sl], o1[s])

    pltpu.sync_copy(vel_hbm, x0)
    pltpu.sync_copy(b_hbm, bias_v)

    @plsc.parallel_loop(0, CAP_CHUNKS, unroll=4)
    def _(c):
        s = pl.ds(c * LANES, LANES)
        j = jiref[s] & (N - 1)
        g = plsc.load_gather(x0, [j])
        vref[s] = vref[s] * g

    xbufs, obufs = (x0, x1), (o0, o1)
    xsems, osems = (sx0, sx1), (so0, so1)

    pltpu.async_copy(x_hbm.at[base_row], x0, sx0)

    def pair_body(it, carry):
        for bslot in range(2):
            r = base_row + it * 2 + bslot
            xb, ob = xbufs[bslot], obufs[bslot]
            xs, os_ = xsems[bslot], osems[bslot]

            @pl.when(it * 2 + bslot + 1 < ROWS_PER_W)
            def _():
                pltpu.async_copy(
                    x_hbm.at[r + 1], xbufs[1 - bslot], xsems[1 - bslot])

            @pl.when(it >= 1)
            def _():
                pltpu.make_async_copy(ob, out_hbm.at[r - 2], os_).wait()

            @plsc.parallel_loop(0, N // LANES, unroll=16)
            def _(c):
                s = pl.ds(c * LANES, LANES)
                ob[s] = bias_v[s]

            pltpu.make_async_copy(x_hbm.at[r], xb, xs).wait()

            @plsc.parallel_loop(0, CAP_CHUNKS, unroll=8)
            def _(c):
                s = pl.ds(c * LANES, LANES)
                ji = jiref[s]
                v = vref[s]
                j = ji & (N - 1)
                i = lax.shift_right_logical(ji, 14)
                g = plsc.load_gather(xb, [j])
                plsc.addupdate_scatter(ob, [i], v * g)

            pltpu.async_copy(ob, out_hbm.at[r], os_)
        return carry

    lax.fori_loop(0, PAIRS_PER_W, pair_body, 0)

    pltpu.make_async_copy(o0, out_hbm.at[base_row + ROWS_PER_W - 2], so0).wait()
    pltpu.make_async_copy(o1, out_hbm.at[base_row + ROWS_PER_W - 1], so1).wait()


_mesh = plsc.VectorSubcoreMesh(core_axis_name="c", subcore_axis_name="s")

_call = functools.partial(
    pl.kernel,
    mesh=_mesh,
    out_type=jax.ShapeDtypeStruct((B, N), jnp.float32),
    compiler_params=pltpu.CompilerParams(needs_layout_passes=False),
    scratch_types=[
        pltpu.VMEM((CAP,), jnp.int32),
        pltpu.VMEM((CAP,), jnp.float32),
        pltpu.VMEM((N,), jnp.float32),
        pltpu.VMEM((N,), jnp.float32),
        pltpu.VMEM((N,), jnp.float32),
        pltpu.VMEM((N,), jnp.float32),
        pltpu.VMEM((N,), jnp.float32),
        pltpu.SemaphoreType.DMA,
        pltpu.SemaphoreType.DMA,
        pltpu.SemaphoreType.DMA,
        pltpu.SemaphoreType.DMA,
    ],
)(_sc_kernel)


def kernel(inputs, W3, b, velocity, I, J):
    jb = J & 15
    d = (I - J) & 15
    bucket = d * 16 + jb
    onehot = (bucket[:, None] == jnp.arange(256, dtype=jnp.int32)[None, :])
    csum = jnp.cumsum(onehot.astype(jnp.int32), axis=0)
    rank = jnp.take_along_axis(csum, bucket[:, None], axis=1)[:, 0] - 1
    counts = csum[-1]
    maxd = counts.reshape(16, 16).max(axis=1)
    based = jnp.cumsum(maxd) - maxd
    chunk_idx = based[d] + rank
    slot_cf = chunk_idx * 16 + jb
    fits = maxd.sum() <= CAP_CHUNKS - 1
    slot = jnp.where(fits, slot_cf, jnp.arange(NNZ, dtype=jnp.int32))

    pad = NNZP - NNZ
    slot_p = jnp.concatenate(
        [slot, CAP - pad + jnp.arange(pad, dtype=jnp.int32)])
    ji_p = jnp.concatenate([I * N + J, jnp.zeros((pad,), jnp.int32)])
    w_p = jnp.concatenate([W3, jnp.zeros((pad,), jnp.float32)])
    return _call(inputs, w_p, b, velocity,
                 lax.bitcast_convert_type(ji_p, jnp.float32),
                 lax.bitcast_convert_type(slot_p, jnp.float32))


# speedup vs baseline: 2.3769x; 2.1004x over previous
import functools

import jax
import jax.numpy as jnp
from jax import lax
from jax.experimental import pallas as pl
from jax.experimental.pallas import tpu as pltpu, tpu_sc as plsc

B = 4096
N = 16384
NNZ = 12300
LANES = 16
NNZP = ((NNZ + LANES - 1) // LANES) * LANES
SRC_CHUNKS = NNZP // LANES
CAP_CHUNKS = 1152
CAP = CAP_CHUNKS * LANES

NUM_CORES = 2
NUM_SUBCORES = 16
NW = NUM_CORES * NUM_SUBCORES
ROWS_PER_W = B // NW
PAIRS_PER_W = ROWS_PER_W // 2

def _sc_kernel(x_hbm, w_hbm, b_hbm, vel_hbm, jif_hbm, slotf_hbm, out_hbm,
               jiref, vref, bias_v, x0, x1, o0, o1,
               sx0, sx1, so0, so1):
    wid = lax.axis_index("s") * NUM_CORES + lax.axis_index("c")
    base_row = wid * ROWS_PER_W

    pltpu.sync_copy(jif_hbm, o0.at[pl.ds(0, NNZP)])
    pltpu.sync_copy(slotf_hbm, x1.at[pl.ds(0, NNZP)])
    pltpu.sync_copy(w_hbm, o1.at[pl.ds(0, NNZP)])

    @plsc.parallel_loop(0, CAP_CHUNKS, unroll=8)
    def _(c):
        s = pl.ds(c * LANES, LANES)
        jiref[s] = lax.iota(jnp.int32, LANES) * (N + 1)
        vref[s] = jnp.zeros((LANES,), jnp.float32)

    @plsc.parallel_loop(0, SRC_CHUNKS, unroll=4)
    def _(c):
        s = pl.ds(c * LANES, LANES)
        sl = plsc.bitcast(x1[s], jnp.int32)
        jiv = plsc.bitcast(o0[s], jnp.int32)
        plsc.store_scatter(jiref, [sl], jiv)
        plsc.store_scatter(vref, [sl], o1[s])

    pltpu.sync_copy(vel_hbm, x0)
    pltpu.sync_copy(b_hbm, bias_v)

    @plsc.parallel_loop(0, CAP_CHUNKS, unroll=4)
    def _(c):
        s = pl.ds(c * LANES, LANES)
        j = jiref[s] & (N - 1)
        g = plsc.load_gather(x0, [j])
        vref[s] = vref[s] * g

    xbufs, obufs = (x0, x1), (o0, o1)
    xsems, osems = (sx0, sx1), (so0, so1)

    pltpu.async_copy(x_hbm.at[base_row], x0, sx0)

    def pair_body(it, carry):
        for bslot in range(2):
            r = base_row + it * 2 + bslot
            xb, ob = xbufs[bslot], obufs[bslot]
            xs, os_ = xsems[bslot], osems[bslot]

            @pl.when(it * 2 + bslot + 1 < ROWS_PER_W)
            def _():
                pltpu.async_copy(
                    x_hbm.at[r + 1], xbufs[1 - bslot], xsems[1 - bslot])

            @pl.when(it >= 1)
            def _():
                pltpu.make_async_copy(ob, out_hbm.at[r - 2], os_).wait()

            @plsc.parallel_loop(0, N // LANES, unroll=16)
            def _(c):
                s = pl.ds(c * LANES, LANES)
                ob[s] = bias_v[s]

            pltpu.make_async_copy(x_hbm.at[r], xb, xs).wait()

            @plsc.parallel_loop(0, CAP_CHUNKS, unroll=8)
            def _(c):
                s = pl.ds(c * LANES, LANES)
                ji = jiref[s]
                v = vref[s]
                j = ji & (N - 1)
                i = lax.shift_right_logical(ji, 14)
                g = plsc.load_gather(xb, [j])
                plsc.addupdate_scatter(ob, [i], v * g)

            pltpu.async_copy(ob, out_hbm.at[r], os_)
        return carry

    lax.fori_loop(0, PAIRS_PER_W, pair_body, 0)

    pltpu.make_async_copy(o0, out_hbm.at[base_row + ROWS_PER_W - 2], so0).wait()
    pltpu.make_async_copy(o1, out_hbm.at[base_row + ROWS_PER_W - 1], so1).wait()


_mesh = plsc.VectorSubcoreMesh(core_axis_name="c", subcore_axis_name="s")

_call = functools.partial(
    pl.kernel,
    mesh=_mesh,
    out_type=jax.ShapeDtypeStruct((B, N), jnp.float32),
    compiler_params=pltpu.CompilerParams(needs_layout_passes=False),
    scratch_types=[
        pltpu.VMEM((CAP,), jnp.int32),
        pltpu.VMEM((CAP,), jnp.float32),
        pltpu.VMEM((N,), jnp.float32),
        pltpu.VMEM((N,), jnp.float32),
        pltpu.VMEM((N,), jnp.float32),
        pltpu.VMEM((N,), jnp.float32),
        pltpu.VMEM((N,), jnp.float32),
        pltpu.SemaphoreType.DMA,
        pltpu.SemaphoreType.DMA,
        pltpu.SemaphoreType.DMA,
        pltpu.SemaphoreType.DMA,
    ],
)(_sc_kernel)


def kernel(inputs, W3, b, velocity, I, J):
    jb = J & 15
    d = (I - J) & 15
    bucket = d * 16 + jb
    onehot = (bucket[:, None] == jnp.arange(256, dtype=jnp.int32)[None, :])
    csum = jnp.cumsum(onehot.astype(jnp.int32), axis=0)
    rank = jnp.take_along_axis(csum, bucket[:, None], axis=1)[:, 0] - 1
    counts = csum[-1]
    maxd = counts.reshape(16, 16).max(axis=1)
    based = jnp.cumsum(maxd) - maxd
    chunk_idx = based[d] + rank
    slot_cf = chunk_idx * 16 + jb
    fits = maxd.sum() <= CAP_CHUNKS - 1
    slot = jnp.where(fits, slot_cf, jnp.arange(NNZ, dtype=jnp.int32))

    pad = NNZP - NNZ
    slot_p = jnp.concatenate(
        [slot, CAP - pad + jnp.arange(pad, dtype=jnp.int32)])
    ji_p = jnp.concatenate([I * N + J, jnp.zeros((pad,), jnp.int32)])
    w_p = jnp.concatenate([W3, jnp.zeros((pad,), jnp.float32)])
    return _call(inputs, w_p, b, velocity,
                 lax.bitcast_convert_type(ji_p, jnp.float32),
                 lax.bitcast_convert_type(slot_p, jnp.float32))


# speedup vs baseline: 3.8813x; 1.6329x over previous
import functools

import jax
import jax.numpy as jnp
from jax import lax
from jax.experimental import pallas as pl
from jax.experimental.pallas import tpu as pltpu, tpu_sc as plsc

B = 4096
N = 16384
NNZ = 12300
LANES = 16
NNZP = ((NNZ + LANES - 1) // LANES) * LANES
CHUNKS = NNZP // LANES

NUM_CORES = 2
NUM_SUBCORES = 16
NW = NUM_CORES * NUM_SUBCORES
ROWS_PER_W = B // NW
PAIRS_PER_W = ROWS_PER_W // 2


def _sc_kernel(x_hbm, w3_hbm, b_hbm, vel_hbm, ji_hbm, out_hbm,
               jiref, vref, bias_v, x0, x1, o0, o1,
               sx0, sx1, so0, so1):
    wid = lax.axis_index("s") * NUM_CORES + lax.axis_index("c")
    base_row = wid * ROWS_PER_W

    pltpu.sync_copy(ji_hbm, jiref)
    pltpu.sync_copy(w3_hbm, vref)
    pltpu.sync_copy(vel_hbm, x0)
    pltpu.sync_copy(b_hbm, bias_v)

    @plsc.parallel_loop(0, CHUNKS, unroll=4)
    def _(c):
        s = pl.ds(c * LANES, LANES)
        j = jiref[s] & (N - 1)
        g = plsc.load_gather(x0, [j])
        vref[s] = vref[s] * g

    xbufs, obufs = (x0, x1), (o0, o1)
    xsems, osems = (sx0, sx1), (so0, so1)

    pltpu.async_copy(x_hbm.at[base_row], x0, sx0)

    def pair_body(it, carry):
        for bslot in range(2):
            r = base_row + it * 2 + bslot
            xb, ob = xbufs[bslot], obufs[bslot]
            xs, os_ = xsems[bslot], osems[bslot]

            pltpu.make_async_copy(x_hbm.at[r], xb, xs).wait()

            @pl.when(it * 2 + bslot + 1 < ROWS_PER_W)
            def _():
                pltpu.async_copy(
                    x_hbm.at[r + 1], xbufs[1 - bslot], xsems[1 - bslot])

            @pl.when(it >= 1)
            def _():
                pltpu.make_async_copy(ob, out_hbm.at[r - 2], os_).wait()

            @plsc.parallel_loop(0, N // LANES, unroll=8)
            def _(c):
                s = pl.ds(c * LANES, LANES)
                ob[s] = bias_v[s]

            @plsc.parallel_loop(0, CHUNKS, unroll=4)
            def _(c):
                s = pl.ds(c * LANES, LANES)
                ji = jiref[s]
                v = vref[s]
                j = ji & (N - 1)
                i = lax.shift_right_logical(ji, 14)
                g = plsc.load_gather(xb, [j])
                plsc.addupdate_scatter(ob, [i], v * g)

            pltpu.async_copy(ob, out_hbm.at[r], os_)
        return carry

    lax.fori_loop(0, PAIRS_PER_W, pair_body, 0)

    pltpu.make_async_copy(o0, out_hbm.at[base_row + ROWS_PER_W - 2], so0).wait()
    pltpu.make_async_copy(o1, out_hbm.at[base_row + ROWS_PER_W - 1], so1).wait()


_mesh = plsc.VectorSubcoreMesh(core_axis_name="c", subcore_axis_name="s")

_call = functools.partial(
    pl.kernel,
    mesh=_mesh,
    out_type=jax.ShapeDtypeStruct((B, N), jnp.float32),
    compiler_params=pltpu.CompilerParams(needs_layout_passes=False),
    scratch_types=[
        pltpu.VMEM((NNZP,), jnp.int32),
        pltpu.VMEM((NNZP,), jnp.float32),
        pltpu.VMEM((N,), jnp.float32),
        pltpu.VMEM((N,), jnp.float32),
        pltpu.VMEM((N,), jnp.float32),
        pltpu.VMEM((N,), jnp.float32),
        pltpu.VMEM((N,), jnp.float32),
        pltpu.SemaphoreType.DMA,
        pltpu.SemaphoreType.DMA,
        pltpu.SemaphoreType.DMA,
        pltpu.SemaphoreType.DMA,
    ],
)(_sc_kernel)


def kernel(inputs, W3, b, velocity, I, J):
    pad = NNZP - NNZ
    ji = I * N + J
    ji_p = jnp.concatenate([ji, jnp.zeros((pad,), jnp.int32)])
    w_p = jnp.concatenate([W3, jnp.zeros((pad,), jnp.float32)])
    return _call(inputs, w_p, b, velocity, ji_p)
